# baseline (speedup 1.0000x reference)
import jax
import jax.numpy as jnp
from jax.experimental import pallas as pl

_OUT_FEATURES = 4096
_SIZE = 26
_LEVELS = 100
_LPAD = 128
_K = _SIZE * _LPAD
_LOW = 0.0
_HIGH = 1.0
_DTILE_BUILD = 1024
_DTILE_MM = 512


def _build_kernel(x_ref, pos_ref, val_ref, w_ref, m_ref):
    vs = (1 - 2 * val_ref[...].astype(jnp.int32)).astype(jnp.bfloat16)
    vs_pad = jnp.concatenate(
        [vs, jnp.zeros((_LPAD - _LEVELS, vs.shape[1]), jnp.bfloat16)], axis=0
    )
    ps = (1 - 2 * pos_ref[...].astype(jnp.int32)).astype(jnp.bfloat16)
    for s in range(_SIZE):
        w_ref[s * _LPAD : (s + 1) * _LPAD, :] = ps[s : s + 1, :] * vs_pad

    @pl.when(pl.program_id(0) == 0)
    def _build_m():
        x = x_ref[...]
        idx = jnp.clip(
            jnp.round((x - _LOW) / (_HIGH - _LOW) * (_LEVELS - 1)), 0, _LEVELS - 1
        ).astype(jnp.int32)
        lanes = jax.lax.broadcasted_iota(jnp.int32, (x.shape[0], _LPAD), 1)
        m_ref[...] = jnp.concatenate(
            [(idx[:, s : s + 1] == lanes) for s in range(_SIZE)], axis=1
        ).astype(jnp.bfloat16)


def _matmul_kernel(m_ref, w_ref, out_ref):
    t = jnp.dot(m_ref[...], w_ref[...], preferred_element_type=jnp.float32)
    out_ref[...] = (t <= 0.0).astype(jnp.uint8)


def kernel(x, position_weight, value_weight):
    batch = x.shape[0]
    n_build = _OUT_FEATURES // _DTILE_BUILD
    w, m = pl.pallas_call(
        _build_kernel,
        grid=(n_build,),
        in_specs=[
            pl.BlockSpec((batch, _SIZE), lambda j: (0, 0)),
            pl.BlockSpec((_SIZE, _DTILE_BUILD), lambda j: (0, j)),
            pl.BlockSpec((_LEVELS, _DTILE_BUILD), lambda j: (0, j)),
        ],
        out_specs=[
            pl.BlockSpec((_K, _DTILE_BUILD), lambda j: (0, j)),
            pl.BlockSpec((batch, _K), lambda j: (0, 0)),
        ],
        out_shape=[
            jax.ShapeDtypeStruct((_K, _OUT_FEATURES), jnp.bfloat16),
            jax.ShapeDtypeStruct((batch, _K), jnp.bfloat16),
        ],
    )(x, position_weight, value_weight)

    n_mm = _OUT_FEATURES // _DTILE_MM
    return pl.pallas_call(
        _matmul_kernel,
        grid=(n_mm,),
        in_specs=[
            pl.BlockSpec((batch, _K), lambda j: (0, 0)),
            pl.BlockSpec((_K, _DTILE_MM), lambda j: (0, j)),
        ],
        out_specs=pl.BlockSpec((batch, _DTILE_MM), lambda j: (0, j)),
        out_shape=jax.ShapeDtypeStruct((batch, _OUT_FEATURES), jnp.uint8),
    )(m, w)


# speedup vs baseline: 1.4189x; 1.4189x over previous
import jax
import jax.numpy as jnp
from jax.experimental import pallas as pl
from jax.experimental.pallas import tpu as pltpu

_OUT_FEATURES = 4096
_SIZE = 26
_LEVELS = 100
_LPAD = 128
_LOW = 0.0
_HIGH = 1.0
_BTILE = 512


def _encode_kernel(x_ref, pos_ref, val_ref, out_ref, w_ref):
    @pl.when(pl.program_id(0) == 0)
    def _build_w():
        vs = (1 - 2 * val_ref[...].astype(jnp.int32)).astype(jnp.bfloat16)
        vs_pad = jnp.concatenate(
            [vs, jnp.zeros((_LPAD - _LEVELS, _OUT_FEATURES), jnp.bfloat16)], axis=0
        )
        ps = (1 - 2 * pos_ref[...].astype(jnp.int32)).astype(jnp.bfloat16)
        for s in range(_SIZE):
            w_ref[s * _LPAD : (s + 1) * _LPAD, :] = ps[s : s + 1, :] * vs_pad

    x = x_ref[...]
    idx = jnp.clip(
        jnp.round((x - _LOW) / (_HIGH - _LOW) * (_LEVELS - 1)), 0, _LEVELS - 1
    ).astype(jnp.int32)
    lanes = jax.lax.broadcasted_iota(jnp.int32, (x.shape[0], _LPAD), 1)
    m = jnp.concatenate(
        [(idx[:, s : s + 1] == lanes) for s in range(_SIZE)], axis=1
    ).astype(jnp.bfloat16)
    for c in range(0, _OUT_FEATURES, 1024):
        t = jnp.dot(
            m, w_ref[:, c : c + 1024], preferred_element_type=jnp.float32
        )
        out_ref[:, c : c + 1024] = (t <= 0.0).astype(jnp.uint8)


def kernel(x, position_weight, value_weight):
    batch = x.shape[0]
    n_b = batch // _BTILE
    return pl.pallas_call(
        _encode_kernel,
        grid=(n_b,),
        in_specs=[
            pl.BlockSpec((_BTILE, _SIZE), lambda i: (i, 0)),
            pl.BlockSpec((_SIZE, _OUT_FEATURES), lambda i: (0, 0)),
            pl.BlockSpec((_LEVELS, _OUT_FEATURES), lambda i: (0, 0)),
        ],
        out_specs=pl.BlockSpec((_BTILE, _OUT_FEATURES), lambda i: (i, 0)),
        out_shape=jax.ShapeDtypeStruct((batch, _OUT_FEATURES), jnp.uint8),
        scratch_shapes=[pltpu.VMEM((_SIZE * _LPAD, _OUT_FEATURES), jnp.bfloat16)],
    )(x, position_weight, value_weight)


# speedup vs baseline: 3.3511x; 2.3617x over previous
import jax
import jax.numpy as jnp
from jax.experimental import pallas as pl

_OUT_FEATURES = 4096
_SIZE = 26
_LEVELS = 100
_LPAD = 128
_K = _SIZE * _LPAD
_NPLANES = 4
_WORDS = _OUT_FEATURES // _NPLANES
_LOW = 0.0
_HIGH = 1.0


def _pack_planes(bits_i32):
    acc = bits_i32[:, :_WORDS]
    for k in range(1, _NPLANES):
        acc = acc | (bits_i32[:, k * _WORDS : (k + 1) * _WORDS] << (5 * k))
    return acc


def _encode_kernel(x_ref, pos_ref, val_ref, out_ref):
    vp = _pack_planes(val_ref[...].astype(jnp.int32))
    vp = jnp.concatenate(
        [vp, jnp.zeros((_LPAD - _LEVELS, _WORDS), jnp.int32)], axis=0
    )
    pp = _pack_planes(pos_ref[...].astype(jnp.int32))
    w2 = jnp.concatenate(
        [(pp[s : s + 1, :] ^ vp).astype(jnp.float32) for s in range(_SIZE)], axis=0
    )

    x = x_ref[...]
    idx = jnp.clip(
        jnp.round((x - _LOW) / (_HIGH - _LOW) * (_LEVELS - 1)), 0, _LEVELS - 1
    ).astype(jnp.int32)
    lanes = jax.lax.broadcasted_iota(jnp.int32, (x.shape[0], _LPAD), 1)
    m = jnp.concatenate(
        [(idx[:, s : s + 1] == lanes) for s in range(_SIZE)], axis=1
    ).astype(jnp.float32)

    counts = jnp.dot(m, w2, preferred_element_type=jnp.float32).astype(jnp.int32)
    for k in range(_NPLANES):
        c = jax.lax.shift_right_logical(counts, 5 * k) & 31
        out_ref[:, k * _WORDS : (k + 1) * _WORDS] = (c >= 13).astype(jnp.uint8)


def kernel(x, position_weight, value_weight):
    batch = x.shape[0]
    return pl.pallas_call(
        _encode_kernel,
        out_shape=jax.ShapeDtypeStruct((batch, _OUT_FEATURES), jnp.uint8),
    )(x, position_weight, value_weight)
